# baseline (speedup 1.0000x reference)
import functools
import jax
import jax.numpy as jnp
from jax import lax
from jax.experimental import pallas as pl
from jax.experimental.pallas import tpu as pltpu, tpu_sc as plsc

BATCH, TIME, CHANNELS = 32, 2048, 1024
KEEP = 16
SUB, LANE = 16, 128
NC = 2
CH = TIME // 16

NEG = jnp.float32(-jnp.inf)
BIG = jnp.int32(TIME)


def _score_body(x_ref, w_ref, o_ref):
    s_col = jax.lax.dot_general(
        x_ref[...], w_ref[...],
        (((1,), (0,)), ((), ())),
        precision=jax.lax.Precision.DEFAULT,
        preferred_element_type=jnp.float32,
    )
    o_ref[...] = s_col.reshape(SUB, LANE)


def _apply_body(x_ref, m_ref, o_ref):
    o_ref[...] = x_ref[...] * m_ref[...]


def _splat_max(v):
    lane = lax.iota(jnp.int32, 16)
    for k in (1, 2, 4, 8):
        v = jnp.maximum(v, v.at[lane ^ k].get(mode="promise_in_bounds"))
    return v


def _splat_min_i32(v):
    lane = lax.iota(jnp.int32, 16)
    for k in (1, 2, 4, 8):
        v = jnp.minimum(v, v.at[lane ^ k].get(mode="promise_in_bounds"))
    return v


@functools.partial(
    pl.kernel,
    mesh=plsc.VectorSubcoreMesh(core_axis_name="c", subcore_axis_name="s"),
    out_type=jax.ShapeDtypeStruct((BATCH, TIME), jnp.float32),
    scratch_types=[
        pltpu.VMEM((TIME,), jnp.float32),
        pltpu.VMEM((TIME,), jnp.float32),
    ],
)
def _topk_mask_sc(s_hbm, out_hbm, sv, mv):
    wid = lax.axis_index("s") * NC + lax.axis_index("c")
    pltpu.sync_copy(s_hbm.at[wid], sv)

    lane = lax.iota(jnp.int32, 16)

    def zero_body(c, carry):
        mv[pl.ds(c * 16, 16)] = jnp.zeros((16,), jnp.float32)
        return carry

    lax.fori_loop(0, CH, zero_body, jnp.int32(0))

    g_prev = jnp.full((16,), BIG, jnp.int32)

    for _ in range(KEEP):
        def max_body(c, m_vec):
            base = pl.ds(c * 16, 16)
            gidx = c * 16 + lane
            mch = mv[base] + jnp.where(gidx == g_prev, 1.0, 0.0)
            mv[base] = mch
            val = jnp.where(mch > 0.5, NEG, sv[base])
            return jnp.maximum(m_vec, val)

        m_vec = lax.fori_loop(0, CH, max_body, jnp.full((16,), NEG))
        m = _splat_max(m_vec)

        def min_body(c, g_vec):
            base = pl.ds(c * 16, 16)
            gidx = c * 16 + lane
            eq = (sv[base] == m) & (mv[base] < 0.5)
            return jnp.minimum(g_vec, jnp.where(eq, gidx, BIG))

        g_vec = lax.fori_loop(0, CH, min_body, jnp.full((16,), BIG, jnp.int32))
        g_prev = _splat_min_i32(g_vec)

    def fin_body(c, carry):
        base = pl.ds(c * 16, 16)
        gidx = c * 16 + lane
        mv[base] = mv[base] + jnp.where(gidx == g_prev, 1.0, 0.0)
        return carry

    lax.fori_loop(0, CH, fin_body, jnp.int32(0))
    pltpu.sync_copy(mv, out_hbm.at[wid])


def kernel(x, attn_W, attn_b):
    del attn_b
    scores = pl.pallas_call(
        _score_body,
        grid=(BATCH,),
        in_specs=[
            pl.BlockSpec((None, TIME, CHANNELS), lambda b: (b, 0, 0)),
            pl.BlockSpec((CHANNELS, 1), lambda b: (0, 0)),
        ],
        out_specs=pl.BlockSpec((None, SUB, LANE), lambda b: (b, 0, 0)),
        out_shape=jax.ShapeDtypeStruct((BATCH, SUB, LANE), jnp.float32),
    )(x, attn_W)

    mask = _topk_mask_sc(scores.reshape(BATCH, TIME))

    return pl.pallas_call(
        _apply_body,
        grid=(BATCH,),
        in_specs=[
            pl.BlockSpec((None, TIME, CHANNELS), lambda b: (b, 0, 0)),
            pl.BlockSpec((None, TIME, 1), lambda b: (b, 0, 0)),
        ],
        out_specs=pl.BlockSpec((None, TIME, CHANNELS), lambda b: (b, 0, 0)),
        out_shape=jax.ShapeDtypeStruct((BATCH, TIME, CHANNELS), x.dtype),
    )(x, mask.reshape(BATCH, TIME, 1))


# speedup vs baseline: 1.5429x; 1.5429x over previous
import jax
import jax.numpy as jnp
from jax.experimental import pallas as pl
from jax.experimental.pallas import tpu as pltpu

BATCH, TIME, CHANNELS = 32, 2048, 1024
KEEP = 16
SUB = 16
LANE = TIME // SUB


def _body(x_ref, w_ref, o_ref):
    xb = x_ref[...]
    x3 = xb.reshape(SUB, LANE, CHANNELS)
    s_col = jax.lax.dot_general(
        xb, w_ref[...],
        (((1,), (0,)), ((), ())),
        precision=jax.lax.Precision.DEFAULT,
        preferred_element_type=jnp.float32,
    )
    s = s_col.reshape(SUB, LANE)

    gidx = (jax.lax.broadcasted_iota(jnp.int32, (SUB, LANE), 0) * LANE
            + jax.lax.broadcasted_iota(jnp.int32, (SUB, LANE), 1))
    big = jnp.int32(TIME)
    neg = jnp.float32(-jnp.inf)

    colrank = jnp.zeros((SUB, LANE), jnp.int32)
    for r in range(1, SUB):
        sr = jnp.concatenate([s[r:], s[:r]], axis=0)
        gr = jnp.concatenate([gidx[r:], gidx[:r]], axis=0)
        gt = (sr > s) | ((sr == s) & (gr < gidx))
        colrank = colrank + gt.astype(jnp.int32)

    ptr = jnp.zeros((1, LANE), jnp.int32)
    cand_v = jnp.max(jnp.where(colrank == 0, s, neg), axis=0, keepdims=True)
    cand_g = jnp.min(jnp.where(colrank == 0, gidx, big), axis=0, keepdims=True)
    for _ in range(KEEP):
        m = jnp.max(cand_v, axis=1, keepdims=True)
        g = jnp.min(jnp.where(cand_v == m, cand_g, big),
                    axis=1, keepdims=True)
        ptr = ptr + (cand_g == g).astype(jnp.int32)
        onehot = colrank == ptr
        cand_v = jnp.max(jnp.where(onehot, s, neg), axis=0, keepdims=True)
        cand_g = jnp.min(jnp.where(onehot, gidx, big), axis=0, keepdims=True)

    diff = ptr.astype(jnp.float32) - colrank.astype(jnp.float32)
    mask = jnp.minimum(jnp.maximum(diff, 0.0), 1.0)
    o_ref[...] = (x3 * mask[:, :, None]).reshape(TIME, CHANNELS)


def kernel(x, attn_W, attn_b):
    del attn_b
    return pl.pallas_call(
        _body,
        grid=(BATCH,),
        in_specs=[
            pl.BlockSpec((None, TIME, CHANNELS), lambda b: (b, 0, 0)),
            pl.BlockSpec((CHANNELS, 1), lambda b: (0, 0)),
        ],
        out_specs=pl.BlockSpec((None, TIME, CHANNELS), lambda b: (b, 0, 0)),
        out_shape=jax.ShapeDtypeStruct((BATCH, TIME, CHANNELS), x.dtype),
        compiler_params=pltpu.CompilerParams(
            dimension_semantics=("parallel",),
        ),
    )(x, attn_W)


# speedup vs baseline: 1.7149x; 1.1115x over previous
import jax
import jax.numpy as jnp
from jax.experimental import pallas as pl
from jax.experimental.pallas import tpu as pltpu

BATCH, TIME, CHANNELS = 32, 2048, 1024
KEEP = 16
SUB = 16
LANE = TIME // SUB


def _body(x_ref, w_ref, o_ref):
    xb = x_ref[...]
    x3 = xb.reshape(SUB, LANE, CHANNELS)
    s_col = jax.lax.dot_general(
        xb, w_ref[...],
        (((1,), (0,)), ((), ())),
        precision=jax.lax.Precision.DEFAULT,
        preferred_element_type=jnp.float32,
    )
    s = s_col.reshape(SUB, LANE)

    gidx = (jax.lax.broadcasted_iota(jnp.int32, (SUB, LANE), 0) * LANE
            + jax.lax.broadcasted_iota(jnp.int32, (SUB, LANE), 1)
            ).astype(jnp.float32)
    big = jnp.float32(TIME)
    neg = jnp.float32(-jnp.inf)
    one = jnp.float32(1.0)
    zero = jnp.float32(0.0)

    colrank = jnp.zeros((SUB, LANE), jnp.float32)
    for r in range(1, SUB):
        sr = jnp.concatenate([s[r:], s[:r]], axis=0)
        gr = jnp.concatenate([gidx[r:], gidx[:r]], axis=0)
        gt = (sr > s) | ((sr == s) & (gr < gidx))
        colrank = colrank + jnp.where(gt, one, zero)

    ptr = jnp.zeros((1, LANE), jnp.float32)
    cand_v = jnp.max(jnp.where(colrank == zero, s, neg), axis=0, keepdims=True)
    cand_g = jnp.min(jnp.where(colrank == zero, gidx, big), axis=0, keepdims=True)
    for _ in range(KEEP):
        m = jnp.max(cand_v, axis=1, keepdims=True)
        g = jnp.min(jnp.where(cand_v == m, cand_g, big),
                    axis=1, keepdims=True)
        ptr = ptr + jnp.where(cand_g == g, one, zero)
        onehot = colrank == ptr
        cand_v = jnp.max(jnp.where(onehot, s, neg), axis=0, keepdims=True)
        cand_g = jnp.min(jnp.where(onehot, gidx, big), axis=0, keepdims=True)

    mask = jnp.minimum(jnp.maximum(ptr - colrank, zero), one)
    o_ref[...] = (x3 * mask[:, :, None]).reshape(TIME, CHANNELS)


def kernel(x, attn_W, attn_b):
    del attn_b
    return pl.pallas_call(
        _body,
        grid=(BATCH,),
        in_specs=[
            pl.BlockSpec((None, TIME, CHANNELS), lambda b: (b, 0, 0)),
            pl.BlockSpec((CHANNELS, 1), lambda b: (0, 0)),
        ],
        out_specs=pl.BlockSpec((None, TIME, CHANNELS), lambda b: (b, 0, 0)),
        out_shape=jax.ShapeDtypeStruct((BATCH, TIME, CHANNELS), x.dtype),
        compiler_params=pltpu.CompilerParams(
            dimension_semantics=("parallel",),
        ),
    )(x, attn_W)
